# baseline (speedup 1.0000x reference)
import jax
import jax.numpy as jnp
from jax.experimental import pallas as pl

_BM = 1024


def _router_body(x_ref, wt_ref, idx_ref, pw_ref):
    logits = jnp.dot(x_ref[...], wt_ref[...],
                     preferred_element_type=jnp.float32)
    m = jnp.max(logits, axis=-1, keepdims=True)
    e = jnp.exp(logits - m)
    s = jnp.sum(e, axis=-1, keepdims=True)
    pw_ref[...] = e / s
    idx_ref[...] = jnp.argmax(logits, axis=-1).astype(jnp.int32)


def kernel(x, W):
    M, K = x.shape
    E = W.shape[0]
    wt = W.T
    grid = (M // _BM,)
    idx, pw = pl.pallas_call(
        _router_body,
        grid=grid,
        in_specs=[
            pl.BlockSpec((_BM, K), lambda i: (i, 0)),
            pl.BlockSpec((K, E), lambda i: (0, 0)),
        ],
        out_specs=[
            pl.BlockSpec((_BM,), lambda i: (i,)),
            pl.BlockSpec((_BM, E), lambda i: (i, 0)),
        ],
        out_shape=[
            jax.ShapeDtypeStruct((M,), jnp.int32),
            jax.ShapeDtypeStruct((M, E), jnp.float32),
        ],
    )(x, wt)
    return idx, pw


# speedup vs baseline: 1.0462x; 1.0462x over previous
import jax
import jax.numpy as jnp
from jax.experimental import pallas as pl

_BM = 1024


def _router_body(x_ref, wt_ref, idx_ref, pw_ref):
    logits = jnp.dot(x_ref[...], wt_ref[...],
                     preferred_element_type=jnp.float32)
    m = jnp.max(logits, axis=-1, keepdims=True)
    e = jnp.exp(logits - m)
    s = jnp.sum(e, axis=-1, keepdims=True)
    pw_ref[...] = e * (1.0 / s)
    iota = jax.lax.broadcasted_iota(jnp.int32, logits.shape, 1)
    idx = jnp.min(jnp.where(logits == m, iota, logits.shape[1]), axis=-1)
    idx_ref[...] = idx


def kernel(x, W):
    M, K = x.shape
    E = W.shape[0]
    wt = W.T
    grid = (M // _BM,)
    idx, pw = pl.pallas_call(
        _router_body,
        grid=grid,
        in_specs=[
            pl.BlockSpec((_BM, K), lambda i: (i, 0)),
            pl.BlockSpec((K, E), lambda i: (0, 0)),
        ],
        out_specs=[
            pl.BlockSpec((_BM,), lambda i: (i,)),
            pl.BlockSpec((_BM, E), lambda i: (i, 0)),
        ],
        out_shape=[
            jax.ShapeDtypeStruct((M,), jnp.int32),
            jax.ShapeDtypeStruct((M, E), jnp.float32),
        ],
    )(x, wt)
    return idx, pw
